# baseline (speedup 1.0000x reference)
import functools

import jax
import jax.numpy as jnp
from jax import lax
from jax.experimental import pallas as pl
from jax.experimental.pallas import tpu as pltpu
from jax.experimental.pallas import tpu_sc as plsc

N = 500
E = 2000
F = 128
NP = 512

NS = 16
EPW = 128


def _gather_body(h_hbm, row_hbm, col_hbm, hrow_hbm, hcol_hbm,
                 idx_r, idx_c, hbuf_r, hbuf_c, sem_r, sem_c):
    wid = lax.axis_index("s")
    base = jnp.minimum(wid * EPW, E - EPW)
    ld_r = pltpu.async_copy(row_hbm.at[pl.ds(base, EPW)], idx_r, sem_r)
    ld_c = pltpu.async_copy(col_hbm.at[pl.ds(base, EPW)], idx_c, sem_c)
    ld_r.wait()
    cp_r = pltpu.async_copy(h_hbm.at[idx_r], hbuf_r, sem_r)
    ld_c.wait()
    cp_c = pltpu.async_copy(h_hbm.at[idx_c], hbuf_c, sem_c)
    cp_r.wait()
    wb_r = pltpu.async_copy(hbuf_r, hrow_hbm.at[pl.ds(base, EPW)], sem_r)
    cp_c.wait()
    wb_c = pltpu.async_copy(hbuf_c, hcol_hbm.at[pl.ds(base, EPW)], sem_c)
    wb_r.wait()
    wb_c.wait()


@functools.cache
def _gather_call():
    mesh = plsc.VectorSubcoreMesh(core_axis_name="c", subcore_axis_name="s",
                                  num_cores=1)
    return pl.kernel(
        _gather_body,
        mesh=mesh,
        out_type=(jax.ShapeDtypeStruct((E, F), jnp.float32),
                  jax.ShapeDtypeStruct((E, F), jnp.float32)),
        scratch_types=[
            pltpu.VMEM((EPW,), jnp.int32),
            pltpu.VMEM((EPW,), jnp.int32),
            pltpu.VMEM((EPW, F), jnp.float32),
            pltpu.VMEM((EPW, F), jnp.float32),
            pltpu.SemaphoreType.DMA,
            pltpu.SemaphoreType.DMA,
        ],
    )


def _dense_body(hrow_ref, hcol_ref, ei_ref,
                h_ref, cv_ref,
                w1_ref, b1_ref, w2_ref, b2_ref,
                wc1_ref, bc1_ref, wc2r_ref, bc2_ref,
                wn1_ref, bn1_ref, wn2_ref, bn2_ref,
                hout_ref, cvout_ref):
    f32 = jnp.float32
    bf16 = jnp.bfloat16
    hr = hrow_ref[...].astype(bf16)
    hc = hcol_ref[...].astype(bf16)

    rowv = ei_ref[0:1, :]
    colv = ei_ref[1:2, :]
    eiota = lax.broadcasted_iota(jnp.int32, (E, NP), 1)
    inc = (jnp.where(eiota == rowv.reshape(E, 1), 1.0, 0.0)
           - jnp.where(eiota == colv.reshape(E, 1), 1.0, 0.0))
    coord = cv_ref[:, :3]
    vel = cv_ref[:, 3:6]
    coordp = jnp.concatenate(
        [coord, jnp.zeros((N, 13), f32)], axis=1)
    coordp = jnp.concatenate(
        [coordp, jnp.zeros((NP - N, 16), f32)], axis=0)
    cd16 = jnp.dot(inc, coordp, preferred_element_type=f32)
    cd = cd16[:, :3]
    radial = jnp.sum(cd * cd, axis=1, keepdims=True)

    x = (jnp.dot(hr, w1_ref[:F, :].astype(bf16), preferred_element_type=f32)
         + jnp.dot(hc, w1_ref[F:2 * F, :].astype(bf16),
                   preferred_element_type=f32)
         + radial * w1_ref[2 * F:2 * F + 1, :]
         + b1_ref[...])
    x = jnp.maximum(x, 0.0).astype(bf16)
    ef = jnp.maximum(
        jnp.dot(x, w2_ref[...].astype(bf16), preferred_element_type=f32)
        + b2_ref[...], 0.0)
    efb = ef.astype(bf16)

    c1 = jnp.maximum(
        jnp.dot(efb, wc1_ref[...].astype(bf16), preferred_element_type=f32)
        + bc1_ref[...], 0.0)
    cm = jnp.sum(c1 * wc2r_ref[...], axis=1, keepdims=True) + bc2_ref[0, 0]
    trans = jnp.clip(cd * cm, -100.0, 100.0)

    ones = jnp.ones((E, 1), bf16)
    payload = jnp.concatenate([efb, trans.astype(bf16), ones], axis=1)

    niota = lax.broadcasted_iota(jnp.int32, (NP, E), 0)
    oh = jnp.where(niota == rowv, 1.0, 0.0).astype(bf16)
    agg = jnp.dot(oh, payload, preferred_element_type=f32)

    aggn = agg[:N, :F]
    ts = agg[:N, F:F + 3]
    cnt = agg[:N, F + 3:F + 4]
    aggc = ts / jnp.maximum(cnt, 1.0)

    v = vel + aggc * 0.125
    cvout_ref[...] = jnp.concatenate(
        [coord + v * 0.125, v, jnp.zeros((N, 2), f32)], axis=1)

    hn = h_ref[...]
    y = jnp.maximum(
        jnp.dot(hn.astype(bf16), wn1_ref[:F, :].astype(bf16),
                preferred_element_type=f32)
        + jnp.dot(aggn.astype(bf16), wn1_ref[F:, :].astype(bf16),
                  preferred_element_type=f32)
        + bn1_ref[...], 0.0)
    hout_ref[...] = (hn
                     + jnp.dot(y.astype(bf16), wn2_ref[...].astype(bf16),
                               preferred_element_type=f32)
                     + bn2_ref[...])


_dense_call = pl.pallas_call(
    _dense_body,
    out_shape=(jax.ShapeDtypeStruct((N, F), jnp.float32),
               jax.ShapeDtypeStruct((N, 8), jnp.float32)),
)


def kernel(h, edge_index, coord, vel, vel_init,
           We1, be1, We2, be2, Wn1, bn1, Wn2, bn2, Wc1, bc1, Wc2, bc2):
    del vel_init
    ei = edge_index.astype(jnp.int32)
    row = ei[0]
    col = ei[1]
    cv = jnp.concatenate(
        [coord, vel, jnp.zeros((N, 2), jnp.float32)], axis=1)

    hrow, hcol = _gather_call()(h, row, col)

    h_new, cvout = _dense_call(
        hrow, hcol, ei,
        h, cv,
        We1, be1[None],
        We2, be2[None], Wc1, bc1[None], Wc2.T, bc2[None],
        Wn1, bn1[None], Wn2, bn2[None])

    return (h_new, cvout[:, :3], cvout[:, 3:6])


# speedup vs baseline: 1.0092x; 1.0092x over previous
import functools

import jax
import jax.numpy as jnp
from jax import lax
from jax.experimental import pallas as pl
from jax.experimental.pallas import tpu as pltpu
from jax.experimental.pallas import tpu_sc as plsc

N = 500
E = 2000
F = 128
NP = 512

NS = 16
EPW = 128


def _gather_body(h_hbm, ei_hbm, hrow_hbm, hcol_hbm,
                 idx_r, idx_c, hbuf_r, hbuf_c, sem_r, sem_c):
    wid = lax.axis_index("s")
    base = jnp.minimum(wid * EPW, E - EPW)
    ld_r = pltpu.async_copy(ei_hbm.at[pl.ds(base, EPW)], idx_r, sem_r)
    ld_c = pltpu.async_copy(ei_hbm.at[pl.ds(E + base, EPW)], idx_c, sem_c)
    ld_r.wait()
    cp_r = pltpu.async_copy(h_hbm.at[idx_r], hbuf_r, sem_r)
    ld_c.wait()
    cp_c = pltpu.async_copy(h_hbm.at[idx_c], hbuf_c, sem_c)
    cp_r.wait()
    wb_r = pltpu.async_copy(hbuf_r, hrow_hbm.at[pl.ds(base, EPW)], sem_r)
    cp_c.wait()
    wb_c = pltpu.async_copy(hbuf_c, hcol_hbm.at[pl.ds(base, EPW)], sem_c)
    wb_r.wait()
    wb_c.wait()


@functools.cache
def _gather_call():
    mesh = plsc.VectorSubcoreMesh(core_axis_name="c", subcore_axis_name="s",
                                  num_cores=1)
    return pl.kernel(
        _gather_body,
        mesh=mesh,
        out_type=(jax.ShapeDtypeStruct((E, F), jnp.float32),
                  jax.ShapeDtypeStruct((E, F), jnp.float32)),
        scratch_types=[
            pltpu.VMEM((EPW,), jnp.int32),
            pltpu.VMEM((EPW,), jnp.int32),
            pltpu.VMEM((EPW, F), jnp.float32),
            pltpu.VMEM((EPW, F), jnp.float32),
            pltpu.SemaphoreType.DMA,
            pltpu.SemaphoreType.DMA,
        ],
    )


def _dense_body(hrow_ref, hcol_ref, ei_ref,
                h_ref, cv_ref,
                w1_ref, b1_ref, w2_ref, b2_ref,
                wc1_ref, bc1_ref, wc2r_ref, bc2_ref,
                wn1_ref, bn1_ref, wn2_ref, bn2_ref,
                hout_ref, cvout_ref):
    f32 = jnp.float32
    bf16 = jnp.bfloat16
    hr = hrow_ref[...].astype(bf16)
    hc = hcol_ref[...].astype(bf16)

    rowv = ei_ref[0:1, :]
    colv = ei_ref[1:2, :]
    eiota = lax.broadcasted_iota(jnp.int32, (E, NP), 1)
    inc = (jnp.where(eiota == rowv.reshape(E, 1), 1.0, 0.0)
           - jnp.where(eiota == colv.reshape(E, 1), 1.0, 0.0))
    coord = cv_ref[:, :3]
    vel = cv_ref[:, 3:6]
    coordp = jnp.concatenate(
        [coord, jnp.zeros((N, 13), f32)], axis=1)
    coordp = jnp.concatenate(
        [coordp, jnp.zeros((NP - N, 16), f32)], axis=0)
    cd16 = jnp.dot(inc, coordp, preferred_element_type=f32)
    cd = cd16[:, :3]
    radial = jnp.sum(cd * cd, axis=1, keepdims=True)

    x = (jnp.dot(hr, w1_ref[:F, :].astype(bf16), preferred_element_type=f32)
         + jnp.dot(hc, w1_ref[F:2 * F, :].astype(bf16),
                   preferred_element_type=f32)
         + radial * w1_ref[2 * F:2 * F + 1, :]
         + b1_ref[...])
    x = jnp.maximum(x, 0.0).astype(bf16)
    ef = jnp.maximum(
        jnp.dot(x, w2_ref[...].astype(bf16), preferred_element_type=f32)
        + b2_ref[...], 0.0)
    efb = ef.astype(bf16)

    c1 = jnp.maximum(
        jnp.dot(efb, wc1_ref[...].astype(bf16), preferred_element_type=f32)
        + bc1_ref[...], 0.0)
    cm = jnp.sum(c1 * wc2r_ref[...], axis=1, keepdims=True) + bc2_ref[0, 0]
    trans = jnp.clip(cd * cm, -100.0, 100.0)

    ones = jnp.ones((E, 1), bf16)
    payload = jnp.concatenate([efb, trans.astype(bf16), ones], axis=1)

    niota = lax.broadcasted_iota(jnp.int32, (NP, E), 0)
    oh = jnp.where(niota == rowv, 1.0, 0.0).astype(bf16)
    agg = jnp.dot(oh, payload, preferred_element_type=f32)

    aggn = agg[:N, :F]
    ts = agg[:N, F:F + 3]
    cnt = agg[:N, F + 3:F + 4]
    aggc = ts / jnp.maximum(cnt, 1.0)

    v = vel + aggc * 0.125
    cvout_ref[...] = jnp.concatenate(
        [coord + v * 0.125, v, jnp.zeros((N, 2), f32)], axis=1)

    hn = h_ref[...]
    y = jnp.maximum(
        jnp.dot(hn.astype(bf16), wn1_ref[:F, :].astype(bf16),
                preferred_element_type=f32)
        + jnp.dot(aggn.astype(bf16), wn1_ref[F:, :].astype(bf16),
                  preferred_element_type=f32)
        + bn1_ref[...], 0.0)
    hout_ref[...] = (hn
                     + jnp.dot(y.astype(bf16), wn2_ref[...].astype(bf16),
                               preferred_element_type=f32)
                     + bn2_ref[...])


_dense_call = pl.pallas_call(
    _dense_body,
    out_shape=(jax.ShapeDtypeStruct((N, F), jnp.float32),
               jax.ShapeDtypeStruct((N, 8), jnp.float32)),
)


def kernel(h, edge_index, coord, vel, vel_init,
           We1, be1, We2, be2, Wn1, bn1, Wn2, bn2, Wc1, bc1, Wc2, bc2):
    del vel_init
    ei = edge_index.astype(jnp.int32)
    cv = jnp.concatenate(
        [coord, vel, jnp.zeros((N, 2), jnp.float32)], axis=1)

    hrow, hcol = _gather_call()(h, ei.reshape(2 * E))

    h_new, cvout = _dense_call(
        hrow, hcol, ei,
        h, cv,
        We1, be1[None],
        We2, be2[None], Wc1, bc1[None], Wc2.T, bc2[None],
        Wn1, bn1[None], Wn2, bn2[None])

    return (h_new, cvout[:, :3], cvout[:, 3:6])
